# baseline (speedup 1.0000x reference)
import functools

import jax
import jax.numpy as jnp
from jax import lax
from jax.experimental import pallas as pl
from jax.experimental.pallas import tpu as pltpu
from jax.experimental.pallas import tpu_sc as plsc

BATCH = 16384
DIM = 64
VOCAB = 1000000

BLK = 65536
R_SC = 1 * BLK
R_TC = VOCAB - R_SC
TC_GRID = -(-R_TC // BLK)

NUM_CORES = 2
NUM_SUBCORES = 16
NW = NUM_CORES * NUM_SUBCORES
B_PER_W = BATCH // NW
N_CHUNK = B_PER_W // 128
LANES = 16

MV_PER_W = R_SC // NW
MV_COLS = 256
MV_CHUNKS2 = MV_PER_W // MV_COLS
N_GRP = DIM // 8


def _mv_body(tab_ref, w_ref, tv_ref):
    tv_ref[...] = jnp.sum(tab_ref[...] * w_ref[...], axis=0)


_tc_matvec = pl.pallas_call(
    _mv_body,
    grid=(TC_GRID,),
    in_specs=[
        pl.BlockSpec((DIM, BLK), lambda i: (0, i + R_SC // BLK)),
        pl.BlockSpec((DIM, 1), lambda i: (0, 0)),
    ],
    out_specs=pl.BlockSpec((BLK,), lambda i: (i,)),
    out_shape=jax.ShapeDtypeStruct((R_TC,), jnp.float32),
)

_mesh = plsc.VectorSubcoreMesh(core_axis_name="c", subcore_axis_name="s")


@functools.partial(
    pl.kernel,
    mesh=_mesh,
    out_type=jax.ShapeDtypeStruct((R_SC,), jnp.float32),
    compiler_params=pltpu.CompilerParams(
        needs_layout_passes=False, use_tc_tiling_on_sc=True),
    scratch_types=[
        pltpu.VMEM((2, 8, MV_COLS), jnp.float32),
        pltpu.VMEM((DIM,), jnp.float32),
        pltpu.VMEM((DIM * LANES,), jnp.float32),
        pltpu.VMEM((MV_PER_W,), jnp.float32),
        pltpu.SemaphoreType.DMA,
        pltpu.SemaphoreType.DMA,
    ],
)
def _sc_matvec(tab_hbm, w_hbm, tv_hbm, blk_v, w_v, wsp_v, y_v, sem0, sem1):
    wid = lax.axis_index("s") * NUM_CORES + lax.axis_index("c")
    base = wid * MV_PER_W

    pltpu.sync_copy(w_hbm, w_v)
    for j in range(4):
        wv = w_v[pl.ds(16 * j, 16)]
        for l in range(LANES):
            wsp_v[pl.ds((16 * j + l) * LANES, LANES)] = lax.gather(
                wv,
                jnp.full((LANES, 1), l, jnp.int32),
                lax.GatherDimensionNumbers(
                    offset_dims=(), collapsed_slice_dims=(0,),
                    start_index_map=(0,)),
                (1,),
                mode=lax.GatherScatterMode.PROMISE_IN_BOUNDS)

    sems = [sem0, sem1]

    def _start(g, c, buf):
        col0 = pl.multiple_of(base + c * MV_COLS, 128)
        pltpu.async_copy(
            tab_hbm.at[pl.ds(8 * g, 8), pl.ds(col0, MV_COLS)],
            blk_v.at[buf], sems[buf])

    for g in range(N_GRP):
        wregs = [wsp_v[pl.ds((8 * g + r) * LANES, LANES)] for r in range(8)]
        if g == 0:
            _start(0, 0, 0)
            _start(0, 1, 1)

        def chunk_pair(p, _, g=g, wregs=wregs):
            for parity in range(2):
                c = 2 * p + parity
                pltpu.make_async_copy(
                    tab_hbm.at[pl.ds(0, 8), pl.ds(0, MV_COLS)],
                    blk_v.at[parity], sems[parity]).wait()
                chunk = blk_v.at[parity]
                for k in range(MV_COLS // LANES):
                    acc = chunk[0, pl.ds(16 * k, 16)] * wregs[0]
                    for r in range(1, 8):
                        acc = acc + chunk[r, pl.ds(16 * k, 16)] * wregs[r]
                    off = pl.ds(c * MV_COLS + 16 * k, 16)
                    if g == 0:
                        y_v[off] = acc
                    else:
                        y_v[off] = y_v[off] + acc

                nxt_c = c + 2
                if g == N_GRP - 1:
                    @pl.when(nxt_c < MV_CHUNKS2)
                    def _():
                        _start(g, nxt_c, parity)
                else:
                    nc = jnp.where(nxt_c < MV_CHUNKS2, nxt_c, nxt_c - MV_CHUNKS2)
                    ng = jnp.where(nxt_c < MV_CHUNKS2, g, g + 1)
                    col0 = pl.multiple_of(base + nc * MV_COLS, 128)
                    row0 = pl.multiple_of(8 * ng, 8)
                    pltpu.async_copy(
                        tab_hbm.at[pl.ds(row0, 8), pl.ds(col0, MV_COLS)],
                        blk_v.at[parity], sems[parity])
            return 0

        lax.fori_loop(0, MV_CHUNKS2 // 2, chunk_pair, 0)

    pltpu.sync_copy(y_v, tv_hbm.at[pl.ds(base, MV_PER_W)])


@functools.partial(
    pl.kernel,
    mesh=_mesh,
    out_type=jax.ShapeDtypeStruct((BATCH,), jnp.float32),
    compiler_params=pltpu.CompilerParams(
        needs_layout_passes=False, use_tc_tiling_on_sc=False),
    scratch_types=[
        pltpu.VMEM((N_CHUNK, 128), jnp.int32),
        pltpu.VMEM((N_CHUNK, 128), jnp.int32),
        pltpu.VMEM((N_CHUNK, 128), jnp.int32),
        pltpu.VMEM((N_CHUNK, 128), jnp.float32),
        pltpu.VMEM((N_CHUNK, 128), jnp.float32),
        pltpu.VMEM((N_CHUNK, 128), jnp.float32),
        pltpu.SemaphoreType.DMA,
    ],
)
def _lookup(idx_hbm, lo_hbm, hi_hbm, out_hbm,
            idx_v, idxl_v, idxh_v, gl_v, gh_v, y_v, sem):
    wid = lax.axis_index("s") * NUM_CORES + lax.axis_index("c")
    base = wid * B_PER_W
    for c in range(N_CHUNK):
        pltpu.sync_copy(idx_hbm.at[pl.ds(base + c * 128, 128)], idx_v.at[c])
    for c in range(N_CHUNK):
        for k in range(8):
            v = idx_v[c, pl.ds(16 * k, 16)]
            in_lo = v < R_SC
            idxl_v[c, pl.ds(16 * k, 16)] = jnp.where(in_lo, v, 0)
            idxh_v[c, pl.ds(16 * k, 16)] = jnp.where(in_lo, 0, v - R_SC)
    copies = []
    for c in range(N_CHUNK):
        copies.append(pltpu.async_copy(lo_hbm.at[idxl_v.at[c]], gl_v.at[c], sem))
        copies.append(pltpu.async_copy(hi_hbm.at[idxh_v.at[c]], gh_v.at[c], sem))
    for cp in copies:
        cp.wait()
    for c in range(N_CHUNK):
        for k in range(8):
            v = idx_v[c, pl.ds(16 * k, 16)]
            y_v[c, pl.ds(16 * k, 16)] = jnp.where(
                v < R_SC,
                gl_v[c, pl.ds(16 * k, 16)],
                gh_v[c, pl.ds(16 * k, 16)])
    for c in range(N_CHUNK):
        pltpu.sync_copy(y_v.at[c], out_hbm.at[pl.ds(base + c * 128, 128)])


@jax.jit
def kernel(indices, table, w):
    idx = indices.astype(jnp.int32)
    tab_t = jnp.transpose(table)
    tv_lo = _sc_matvec(tab_t, jnp.reshape(w, (DIM,)))
    tv_hi = _tc_matvec(tab_t, w)
    y = _lookup(idx, tv_lo, tv_hi)
    return jnp.reshape(y, (BATCH, 1))


# speedup vs baseline: 1.7004x; 1.7004x over previous
import functools

import jax
import jax.numpy as jnp
from jax import lax
from jax.experimental import pallas as pl
from jax.experimental.pallas import tpu as pltpu
from jax.experimental.pallas import tpu_sc as plsc

BATCH = 16384
DIM = 64
VOCAB = 1000000

BLK = 65536
GRID = -(-VOCAB // BLK)

NUM_CORES = 2
NUM_SUBCORES = 16
NW = NUM_CORES * NUM_SUBCORES
B_PER_W = BATCH // NW
N_CHUNK = B_PER_W // 128


def _mv_body(tab_ref, w_ref, tv_ref):
    tv_ref[...] = jnp.sum(tab_ref[...] * w_ref[...], axis=0)


_matvec = pl.pallas_call(
    _mv_body,
    grid=(GRID,),
    in_specs=[
        pl.BlockSpec((DIM, BLK), lambda i: (0, i)),
        pl.BlockSpec((DIM, 1), lambda i: (0, 0)),
    ],
    out_specs=pl.BlockSpec((BLK,), lambda i: (i,)),
    out_shape=jax.ShapeDtypeStruct((VOCAB,), jnp.float32),
)

_mesh = plsc.VectorSubcoreMesh(core_axis_name="c", subcore_axis_name="s")


@functools.partial(
    pl.kernel,
    mesh=_mesh,
    out_type=jax.ShapeDtypeStruct((BATCH,), jnp.float32),
    compiler_params=pltpu.CompilerParams(
        needs_layout_passes=False, use_tc_tiling_on_sc=False),
    scratch_types=[
        pltpu.VMEM((N_CHUNK, 128), jnp.int32),
        pltpu.VMEM((N_CHUNK, 128), jnp.float32),
        pltpu.SemaphoreType.DMA,
    ],
)
def _lookup(idx_hbm, tv_hbm, out_hbm, idx_v, g_v, sem):
    wid = lax.axis_index("s") * NUM_CORES + lax.axis_index("c")
    base = wid * B_PER_W
    for c in range(N_CHUNK):
        pltpu.sync_copy(idx_hbm.at[pl.ds(base + c * 128, 128)], idx_v.at[c])
    copies = [
        pltpu.async_copy(tv_hbm.at[idx_v.at[c]], g_v.at[c], sem)
        for c in range(N_CHUNK)
    ]
    for c in range(N_CHUNK):
        copies[c].wait()
        pltpu.sync_copy(g_v.at[c], out_hbm.at[pl.ds(base + c * 128, 128)])


@jax.jit
def kernel(indices, table, w):
    idx = indices.astype(jnp.int32)
    tv = _matvec(jnp.transpose(table), w)
    y = _lookup(idx, tv)
    return jnp.reshape(y, (BATCH, 1))
